# baseline (speedup 1.0000x reference)
import functools

import jax
import jax.numpy as jnp
from jax import lax
from jax.experimental import pallas as pl
from jax.experimental.pallas import tpu as pltpu
from jax.experimental.pallas import tpu_sc as plsc

N_NODES = 10000
D_FEAT = 128
OUT_DIM = 64
N_EDGES = 320000

NC, NS = 2, 16
NW = NC * NS
CHUNK = 128
EPW = 10240
E_PAD = NW * EPW
N_PAD = 10240
RPT = N_PAD // NS
N_CHUNKS = EPW // CHUNK
TOT_CHUNKS = E_PAD // CHUNK
CH0 = 160
CH1 = TOT_CHUNKS // NS - CH0

_MESH = plsc.VectorSubcoreMesh(core_axis_name="c", subcore_axis_name="s")


def _make_spmm(d):
    @functools.partial(
        pl.kernel,
        out_type=jax.ShapeDtypeStruct((NC, N_PAD, d), jnp.float32),
        mesh=_MESH,
        scratch_types=[
            pltpu.VMEM((2, CHUNK), jnp.int32),
            pltpu.VMEM((2, CHUNK), jnp.int32),
            pltpu.VMEM((CHUNK, d), jnp.float32),
            pltpu.VMEM((CHUNK, d), jnp.float32),
            pltpu.VMEM_SHARED((N_PAD, d), jnp.float32),
            pltpu.SemaphoreType.DMA,
            pltpu.SemaphoreType.DMA,
            pltpu.SemaphoreType.DMA,
            pltpu.SemaphoreType.DMA,
        ],
    )
    def spmm(rows_hbm, edges_hbm, zeros_hbm, out_hbm,
             e0, e1, buf0, buf1, agg, g0, g1, i0, i1):
        c = lax.axis_index("c")
        s = lax.axis_index("s")
        r0 = s * RPT
        pltpu.sync_copy(zeros_hbm.at[pl.ds(r0, RPT)], agg.at[pl.ds(r0, RPT)])
        plsc.subcore_barrier()

        base = jnp.where(c == 0, s * CH0, NS * CH0 + s * CH1)
        base = jnp.minimum(base, TOT_CHUNKS - 2)
        iters = jnp.where(c == 0, CH0 // 2, CH1 // 2)
        last = base + 2 * iters - 1
        pltpu.async_copy(edges_hbm.at[base], e0, i0)
        pltpu.async_copy(edges_hbm.at[base + 1], e1, i1)
        pltpu.make_async_copy(edges_hbm.at[base], e0, i0).wait()
        pltpu.async_copy(rows_hbm.at[e0.at[0]], buf0, g0)

        def body(i, carry):
            c0 = base + 2 * i
            pltpu.make_async_copy(edges_hbm.at[base], e1, i1).wait()
            pltpu.async_copy(rows_hbm.at[e1.at[0]], buf1, g1)
            pltpu.make_async_copy(rows_hbm.at[e0.at[0]], buf0, g0).wait()
            pltpu.sync_copy(buf0, agg.at[e0.at[1]], add=True)
            c2 = jnp.minimum(c0 + 2, last)
            pltpu.async_copy(edges_hbm.at[c2], e0, i0)
            pltpu.make_async_copy(edges_hbm.at[base], e0, i0).wait()
            pltpu.async_copy(rows_hbm.at[e0.at[0]], buf0, g0)
            pltpu.make_async_copy(rows_hbm.at[e1.at[0]], buf1, g1).wait()
            pltpu.sync_copy(buf1, agg.at[e1.at[1]], add=True)
            c3 = jnp.minimum(c0 + 3, last)
            pltpu.async_copy(edges_hbm.at[c3], e1, i1)
            return carry

        lax.fori_loop(0, iters, body, 0)
        pltpu.make_async_copy(rows_hbm.at[e0.at[0]], buf0, g0).wait()
        pltpu.make_async_copy(edges_hbm.at[base], e1, i1).wait()
        plsc.subcore_barrier()
        pltpu.sync_copy(agg.at[pl.ds(r0, RPT)], out_hbm.at[c, pl.ds(r0, RPT)])

    return spmm


_spmm128 = _make_spmm(D_FEAT)


@functools.partial(
    pl.kernel,
    out_type=jax.ShapeDtypeStruct((NC, N_PAD, D_FEAT), jnp.float32),
    mesh=_MESH,
    scratch_types=[
        pltpu.VMEM((CHUNK,), jnp.int32),
        pltpu.VMEM((CHUNK, D_FEAT), jnp.float32),
        pltpu.VMEM_SHARED((N_PAD, D_FEAT), jnp.float32),
    ],
)
def _deg_kernel(dst_hbm, ones_hbm, zeros_hbm, out_hbm, idx_d, ones_v, agg):
    c = lax.axis_index("c")
    s = lax.axis_index("s")
    wid = s * NC + c
    r0 = s * RPT
    pltpu.sync_copy(ones_hbm, ones_v)
    pltpu.sync_copy(zeros_hbm.at[pl.ds(r0, RPT)], agg.at[pl.ds(r0, RPT)])
    plsc.subcore_barrier()

    def body(j, carry):
        base = wid * EPW + j * CHUNK
        pltpu.sync_copy(dst_hbm.at[pl.ds(base, CHUNK)], idx_d)
        pltpu.sync_copy(ones_v, agg.at[idx_d], add=True)
        return carry

    lax.fori_loop(0, N_CHUNKS, body, 0)
    plsc.subcore_barrier()
    pltpu.sync_copy(agg.at[pl.ds(r0, RPT)], out_hbm.at[c, pl.ds(r0, RPT)])


_BM = 1000


def _tc_first(degA, degB, x, W1):
    def body(da, db, xb, w, hw_ref, dis_ref):
        deg = da[:, :1] + db[:, :1] + 1.0
        dis = lax.rsqrt(jnp.maximum(deg, 1.0))
        hw = jnp.dot(xb[...], w[...], preferred_element_type=jnp.float32)
        hw_ref[...] = dis * hw
        dis_ref[...] = dis

    return pl.pallas_call(
        body,
        grid=(N_NODES // _BM,),
        in_specs=[
            pl.BlockSpec((_BM, D_FEAT), lambda i: (i, 0)),
            pl.BlockSpec((_BM, D_FEAT), lambda i: (i, 0)),
            pl.BlockSpec((_BM, D_FEAT), lambda i: (i, 0)),
            pl.BlockSpec((D_FEAT, D_FEAT), lambda i: (0, 0)),
        ],
        out_specs=[
            pl.BlockSpec((_BM, D_FEAT), lambda i: (i, 0)),
            pl.BlockSpec((_BM, 1), lambda i: (i, 0)),
        ],
        out_shape=[
            jax.ShapeDtypeStruct((N_NODES, D_FEAT), jnp.float32),
            jax.ShapeDtypeStruct((N_NODES, 1), jnp.float32),
        ],
    )(degA, degB, x, W1)


def _tc_mid(pA, pB, hwp, dis, W, d_out):
    def body(pa, pb, hw, ds, w, out_ref):
        h = jnp.maximum(ds[...] * (pa[...] + pb[...] + hw[...]), 0.0)
        out_ref[...] = ds[...] * jnp.dot(
            h, w[...], preferred_element_type=jnp.float32)

    d_in = hwp.shape[1]
    return pl.pallas_call(
        body,
        grid=(N_NODES // _BM,),
        in_specs=[
            pl.BlockSpec((_BM, d_in), lambda i: (i, 0)),
            pl.BlockSpec((_BM, d_in), lambda i: (i, 0)),
            pl.BlockSpec((_BM, d_in), lambda i: (i, 0)),
            pl.BlockSpec((_BM, 1), lambda i: (i, 0)),
            pl.BlockSpec((d_in, d_out), lambda i: (0, 0)),
        ],
        out_specs=pl.BlockSpec((_BM, d_out), lambda i: (i, 0)),
        out_shape=jax.ShapeDtypeStruct((N_NODES, d_out), jnp.float32),
    )(pA, pB, hwp, dis, W)


def _tc_final(pA, pB, hwp, dis):
    def body(pa, pb, hw, ds, out_ref):
        out_ref[...] = ds[...] * (
            pa[:, :OUT_DIM] + pb[:, :OUT_DIM] + hw[:, :OUT_DIM])

    return pl.pallas_call(
        body,
        grid=(N_NODES // _BM,),
        in_specs=[
            pl.BlockSpec((_BM, D_FEAT), lambda i: (i, 0)),
            pl.BlockSpec((_BM, D_FEAT), lambda i: (i, 0)),
            pl.BlockSpec((_BM, D_FEAT), lambda i: (i, 0)),
            pl.BlockSpec((_BM, 1), lambda i: (i, 0)),
        ],
        out_specs=pl.BlockSpec((_BM, OUT_DIM), lambda i: (i, 0)),
        out_shape=jax.ShapeDtypeStruct((N_NODES, OUT_DIM), jnp.float32),
    )(pA, pB, hwp, dis)


def kernel(x, edge_index, W1, W2, W3):
    src = edge_index[0].astype(jnp.int32)
    dst = edge_index[1].astype(jnp.int32)
    pad = E_PAD - N_EDGES
    srcp = jnp.concatenate([src, jnp.zeros((pad,), jnp.int32)])
    dstp = jnp.concatenate([dst, jnp.full((pad,), N_NODES, jnp.int32)])
    edges3 = jnp.stack(
        [srcp.reshape(TOT_CHUNKS, CHUNK), dstp.reshape(TOT_CHUNKS, CHUNK)],
        axis=1)
    zeros128 = jnp.zeros((N_PAD, D_FEAT), jnp.float32)
    ones128 = jnp.ones((CHUNK, D_FEAT), jnp.float32)
    W3p = jnp.pad(W3, ((0, 0), (0, D_FEAT - OUT_DIM)))

    degP = _deg_kernel(dstp, ones128, zeros128)
    hw1, dis = _tc_first(degP[0], degP[1], x, W1)
    P1 = _spmm128(hw1, edges3, zeros128)
    hw2 = _tc_mid(P1[0], P1[1], hw1, dis, W2, D_FEAT)
    P2 = _spmm128(hw2, edges3, zeros128)
    hw3 = _tc_mid(P2[0], P2[1], hw2, dis, W3p, D_FEAT)
    P3 = _spmm128(hw3, edges3, zeros128)
    return _tc_final(P3[0], P3[1], hw3, dis)


# speedup vs baseline: 2.2447x; 2.2447x over previous
import functools

import jax
import jax.numpy as jnp
from jax import lax
from jax.experimental import pallas as pl
from jax.experimental.pallas import tpu as pltpu
from jax.experimental.pallas import tpu_sc as plsc

N_NODES = 10000
D_FEAT = 128
OUT_DIM = 64
N_EDGES = 320000

NC, NS = 2, 16
NW = NC * NS
CHUNK = 120
TOT_CHUNKS = 2688
E_PAD = TOT_CHUNKS * CHUNK
EPW = E_PAD // NW
N_PAD = 10240
RPT = N_PAD // NS
N_CHUNKS = EPW // CHUNK
CH0 = 120
CH1 = TOT_CHUNKS // NS - CH0

_MESH = plsc.VectorSubcoreMesh(core_axis_name="c", subcore_axis_name="s")


def _make_spmm(d):
    @functools.partial(
        pl.kernel,
        out_type=jax.ShapeDtypeStruct((NC, N_PAD, d), jnp.float32),
        mesh=_MESH,
        scratch_types=[
            pltpu.VMEM((2, CHUNK), jnp.int32),
            pltpu.VMEM((2, CHUNK), jnp.int32),
            pltpu.VMEM((2, CHUNK), jnp.int32),
            pltpu.VMEM((CHUNK, d), jnp.float32),
            pltpu.VMEM((CHUNK, d), jnp.float32),
            pltpu.VMEM((CHUNK, d), jnp.float32),
            pltpu.VMEM_SHARED((N_PAD, d), jnp.float32),
            pltpu.SemaphoreType.DMA,
            pltpu.SemaphoreType.DMA,
            pltpu.SemaphoreType.DMA,
            pltpu.SemaphoreType.DMA,
            pltpu.SemaphoreType.DMA,
            pltpu.SemaphoreType.DMA,
        ],
    )
    def spmm(rows_hbm, edges_hbm, zeros_hbm, out_hbm,
             e0, e1, e2, buf0, buf1, buf2, agg, g0, g1, g2, i0, i1, i2):
        c = lax.axis_index("c")
        s = lax.axis_index("s")
        r0 = s * RPT
        es = (e0, e1, e2)
        bufs = (buf0, buf1, buf2)
        gs = (g0, g1, g2)
        isems = (i0, i1, i2)
        pltpu.sync_copy(zeros_hbm.at[pl.ds(r0, RPT)], agg.at[pl.ds(r0, RPT)])
        plsc.subcore_barrier()

        base = jnp.where(c == 0, s * CH0, NS * CH0 + s * CH1)
        base = jnp.minimum(base, TOT_CHUNKS - 3)
        iters = jnp.where(c == 0, CH0 // 3, CH1 // 3)
        last = base + 3 * iters - 1
        last = jnp.maximum(last, base + 2)

        for k in range(3):
            pltpu.async_copy(edges_hbm.at[base + k], es[k], isems[k])
        for k in range(3):
            pltpu.make_async_copy(edges_hbm.at[base], es[k], isems[k]).wait()
            pltpu.async_copy(rows_hbm.at[es[k].at[0]], bufs[k], gs[k])

        def body(i, carry):
            cb = base + 3 * i
            for k in range(3):
                pltpu.make_async_copy(
                    rows_hbm.at[es[k].at[0]], bufs[k], gs[k]).wait()
                pltpu.sync_copy(bufs[k], agg.at[es[k].at[1]], add=True)
                cn = jnp.minimum(cb + k + 3, last)
                pltpu.async_copy(edges_hbm.at[cn], es[k], isems[k])
                pltpu.make_async_copy(edges_hbm.at[base], es[k], isems[k]).wait()
                pltpu.async_copy(rows_hbm.at[es[k].at[0]], bufs[k], gs[k])
            return carry

        lax.fori_loop(0, iters, body, 0)
        for k in range(3):
            pltpu.make_async_copy(rows_hbm.at[es[k].at[0]], bufs[k], gs[k]).wait()
        plsc.subcore_barrier()
        pltpu.sync_copy(agg.at[pl.ds(r0, RPT)], out_hbm.at[c, pl.ds(r0, RPT)])

    return spmm


_spmm128 = _make_spmm(D_FEAT)


@functools.partial(
    pl.kernel,
    out_type=jax.ShapeDtypeStruct((NC, N_PAD, D_FEAT), jnp.float32),
    mesh=_MESH,
    scratch_types=[
        pltpu.VMEM((CHUNK,), jnp.int32),
        pltpu.VMEM((CHUNK, D_FEAT), jnp.float32),
        pltpu.VMEM_SHARED((N_PAD, D_FEAT), jnp.float32),
    ],
)
def _deg_kernel(dst_hbm, ones_hbm, zeros_hbm, out_hbm, idx_d, ones_v, agg):
    c = lax.axis_index("c")
    s = lax.axis_index("s")
    wid = s * NC + c
    r0 = s * RPT
    pltpu.sync_copy(ones_hbm, ones_v)
    pltpu.sync_copy(zeros_hbm.at[pl.ds(r0, RPT)], agg.at[pl.ds(r0, RPT)])
    plsc.subcore_barrier()

    def body(j, carry):
        base = wid * EPW + j * CHUNK
        pltpu.sync_copy(dst_hbm.at[pl.ds(base, CHUNK)], idx_d)
        pltpu.sync_copy(ones_v, agg.at[idx_d], add=True)
        return carry

    lax.fori_loop(0, N_CHUNKS, body, 0)
    plsc.subcore_barrier()
    pltpu.sync_copy(agg.at[pl.ds(r0, RPT)], out_hbm.at[c, pl.ds(r0, RPT)])


_BM = 1000


def _tc_first(degA, degB, x, W1):
    def body(da, db, xb, w, hw_ref, dis_ref):
        deg = da[:, :1] + db[:, :1] + 1.0
        dis = lax.rsqrt(jnp.maximum(deg, 1.0))
        hw = jnp.dot(xb[...], w[...], preferred_element_type=jnp.float32)
        hw_ref[...] = dis * hw
        dis_ref[...] = dis

    return pl.pallas_call(
        body,
        grid=(N_NODES // _BM,),
        in_specs=[
            pl.BlockSpec((_BM, D_FEAT), lambda i: (i, 0)),
            pl.BlockSpec((_BM, D_FEAT), lambda i: (i, 0)),
            pl.BlockSpec((_BM, D_FEAT), lambda i: (i, 0)),
            pl.BlockSpec((D_FEAT, D_FEAT), lambda i: (0, 0)),
        ],
        out_specs=[
            pl.BlockSpec((_BM, D_FEAT), lambda i: (i, 0)),
            pl.BlockSpec((_BM, 1), lambda i: (i, 0)),
        ],
        out_shape=[
            jax.ShapeDtypeStruct((N_NODES, D_FEAT), jnp.float32),
            jax.ShapeDtypeStruct((N_NODES, 1), jnp.float32),
        ],
    )(degA, degB, x, W1)


def _tc_mid(pA, pB, hwp, dis, W, d_out):
    def body(pa, pb, hw, ds, w, out_ref):
        h = jnp.maximum(ds[...] * (pa[...] + pb[...] + hw[...]), 0.0)
        out_ref[...] = ds[...] * jnp.dot(
            h, w[...], preferred_element_type=jnp.float32)

    d_in = hwp.shape[1]
    return pl.pallas_call(
        body,
        grid=(N_NODES // _BM,),
        in_specs=[
            pl.BlockSpec((_BM, d_in), lambda i: (i, 0)),
            pl.BlockSpec((_BM, d_in), lambda i: (i, 0)),
            pl.BlockSpec((_BM, d_in), lambda i: (i, 0)),
            pl.BlockSpec((_BM, 1), lambda i: (i, 0)),
            pl.BlockSpec((d_in, d_out), lambda i: (0, 0)),
        ],
        out_specs=pl.BlockSpec((_BM, d_out), lambda i: (i, 0)),
        out_shape=jax.ShapeDtypeStruct((N_NODES, d_out), jnp.float32),
    )(pA, pB, hwp, dis, W)


def _tc_final(pA, pB, hwp, dis):
    def body(pa, pb, hw, ds, out_ref):
        out_ref[...] = ds[...] * (
            pa[:, :OUT_DIM] + pb[:, :OUT_DIM] + hw[:, :OUT_DIM])

    return pl.pallas_call(
        body,
        grid=(N_NODES // _BM,),
        in_specs=[
            pl.BlockSpec((_BM, D_FEAT), lambda i: (i, 0)),
            pl.BlockSpec((_BM, D_FEAT), lambda i: (i, 0)),
            pl.BlockSpec((_BM, D_FEAT), lambda i: (i, 0)),
            pl.BlockSpec((_BM, 1), lambda i: (i, 0)),
        ],
        out_specs=pl.BlockSpec((_BM, OUT_DIM), lambda i: (i, 0)),
        out_shape=jax.ShapeDtypeStruct((N_NODES, OUT_DIM), jnp.float32),
    )(pA, pB, hwp, dis)


def kernel(x, edge_index, W1, W2, W3):
    src = edge_index[0].astype(jnp.int32)
    dst = edge_index[1].astype(jnp.int32)
    pad = E_PAD - N_EDGES
    srcp = jnp.concatenate([src, jnp.zeros((pad,), jnp.int32)])
    dstp = jnp.concatenate([dst, jnp.full((pad,), N_NODES, jnp.int32)])
    edges3 = jnp.stack(
        [srcp.reshape(TOT_CHUNKS, CHUNK), dstp.reshape(TOT_CHUNKS, CHUNK)],
        axis=1)
    zeros128 = jnp.zeros((N_PAD, D_FEAT), jnp.float32)
    ones128 = jnp.ones((CHUNK, D_FEAT), jnp.float32)
    W3p = jnp.pad(W3, ((0, 0), (0, D_FEAT - OUT_DIM)))

    degP = _deg_kernel(dstp, ones128, zeros128)
    hw1, dis = _tc_first(degP[0], degP[1], x, W1)
    P1 = _spmm128(hw1, edges3, zeros128)
    hw2 = _tc_mid(P1[0], P1[1], hw1, dis, W2, D_FEAT)
    P2 = _spmm128(hw2, edges3, zeros128)
    hw3 = _tc_mid(P2[0], P2[1], hw2, dis, W3p, D_FEAT)
    P3 = _spmm128(hw3, edges3, zeros128)
    return _tc_final(P3[0], P3[1], hw3, dis)


# speedup vs baseline: 2.2849x; 1.0179x over previous
import functools

import jax
import jax.numpy as jnp
from jax import lax
from jax.experimental import pallas as pl
from jax.experimental.pallas import tpu as pltpu
from jax.experimental.pallas import tpu_sc as plsc

N_NODES = 10000
D_FEAT = 128
OUT_DIM = 64
N_EDGES = 320000

NC, NS = 2, 16
NW = NC * NS
CHUNK = 120
TOT_CHUNKS = 2688
E_PAD = TOT_CHUNKS * CHUNK
EPW = E_PAD // NW
N_PAD = 10240
RPT = N_PAD // NS
N_CHUNKS = EPW // CHUNK
CH0 = 129
CH1 = TOT_CHUNKS // NS - CH0

_MESH = plsc.VectorSubcoreMesh(core_axis_name="c", subcore_axis_name="s")


def _make_spmm(d):
    @functools.partial(
        pl.kernel,
        out_type=jax.ShapeDtypeStruct((NC, N_PAD, d), jnp.float32),
        mesh=_MESH,
        scratch_types=[
            pltpu.VMEM((2, CHUNK), jnp.int32),
            pltpu.VMEM((2, CHUNK), jnp.int32),
            pltpu.VMEM((2, CHUNK), jnp.int32),
            pltpu.VMEM((CHUNK, d), jnp.float32),
            pltpu.VMEM((CHUNK, d), jnp.float32),
            pltpu.VMEM((CHUNK, d), jnp.float32),
            pltpu.VMEM_SHARED((N_PAD, d), jnp.float32),
            pltpu.SemaphoreType.DMA,
            pltpu.SemaphoreType.DMA,
            pltpu.SemaphoreType.DMA,
            pltpu.SemaphoreType.DMA,
            pltpu.SemaphoreType.DMA,
            pltpu.SemaphoreType.DMA,
        ],
    )
    def spmm(rows_hbm, edges_hbm, zeros_hbm, out_hbm,
             e0, e1, e2, buf0, buf1, buf2, agg, g0, g1, g2, i0, i1, i2):
        c = lax.axis_index("c")
        s = lax.axis_index("s")
        r0 = s * RPT
        es = (e0, e1, e2)
        bufs = (buf0, buf1, buf2)
        gs = (g0, g1, g2)
        isems = (i0, i1, i2)
        pltpu.sync_copy(zeros_hbm.at[pl.ds(r0, RPT)], agg.at[pl.ds(r0, RPT)])
        plsc.subcore_barrier()

        base = jnp.where(c == 0, s * CH0, NS * CH0 + s * CH1)
        base = jnp.minimum(base, TOT_CHUNKS - 3)
        iters = jnp.where(c == 0, CH0 // 3, CH1 // 3)
        last = base + 3 * iters - 1
        last = jnp.maximum(last, base + 2)

        for k in range(3):
            pltpu.async_copy(edges_hbm.at[base + k], es[k], isems[k])
        for k in range(3):
            pltpu.make_async_copy(edges_hbm.at[base], es[k], isems[k]).wait()
            pltpu.async_copy(rows_hbm.at[es[k].at[0]], bufs[k], gs[k])

        def body(i, carry):
            cb = base + 3 * i
            for k in range(3):
                pltpu.make_async_copy(
                    rows_hbm.at[es[k].at[0]], bufs[k], gs[k]).wait()
                pltpu.sync_copy(bufs[k], agg.at[es[k].at[1]], add=True)
                cn = jnp.minimum(cb + k + 3, last)
                pltpu.async_copy(edges_hbm.at[cn], es[k], isems[k])
                pltpu.make_async_copy(edges_hbm.at[base], es[k], isems[k]).wait()
                pltpu.async_copy(rows_hbm.at[es[k].at[0]], bufs[k], gs[k])
            return carry

        lax.fori_loop(0, iters, body, 0)
        for k in range(3):
            pltpu.make_async_copy(rows_hbm.at[es[k].at[0]], bufs[k], gs[k]).wait()
        plsc.subcore_barrier()
        pltpu.sync_copy(agg.at[pl.ds(r0, RPT)], out_hbm.at[c, pl.ds(r0, RPT)])

    return spmm


_spmm128 = _make_spmm(D_FEAT)


@functools.partial(
    pl.kernel,
    out_type=jax.ShapeDtypeStruct((NC, N_PAD, D_FEAT), jnp.float32),
    mesh=_MESH,
    scratch_types=[
        pltpu.VMEM((CHUNK,), jnp.int32),
        pltpu.VMEM((CHUNK, D_FEAT), jnp.float32),
        pltpu.VMEM_SHARED((N_PAD, D_FEAT), jnp.float32),
    ],
)
def _deg_kernel(dst_hbm, ones_hbm, zeros_hbm, out_hbm, idx_d, ones_v, agg):
    c = lax.axis_index("c")
    s = lax.axis_index("s")
    wid = s * NC + c
    r0 = s * RPT
    pltpu.sync_copy(ones_hbm, ones_v)
    pltpu.sync_copy(zeros_hbm.at[pl.ds(r0, RPT)], agg.at[pl.ds(r0, RPT)])
    plsc.subcore_barrier()

    def body(j, carry):
        base = wid * EPW + j * CHUNK
        pltpu.sync_copy(dst_hbm.at[pl.ds(base, CHUNK)], idx_d)
        pltpu.sync_copy(ones_v, agg.at[idx_d], add=True)
        return carry

    lax.fori_loop(0, N_CHUNKS, body, 0)
    plsc.subcore_barrier()
    pltpu.sync_copy(agg.at[pl.ds(r0, RPT)], out_hbm.at[c, pl.ds(r0, RPT)])


_BM = 1000


def _tc_first(degA, degB, x, W1):
    def body(da, db, xb, w, hw_ref, dis_ref):
        deg = da[:, :1] + db[:, :1] + 1.0
        dis = lax.rsqrt(jnp.maximum(deg, 1.0))
        hw = jnp.dot(xb[...], w[...], preferred_element_type=jnp.float32)
        hw_ref[...] = dis * hw
        dis_ref[...] = dis

    return pl.pallas_call(
        body,
        grid=(N_NODES // _BM,),
        in_specs=[
            pl.BlockSpec((_BM, D_FEAT), lambda i: (i, 0)),
            pl.BlockSpec((_BM, D_FEAT), lambda i: (i, 0)),
            pl.BlockSpec((_BM, D_FEAT), lambda i: (i, 0)),
            pl.BlockSpec((D_FEAT, D_FEAT), lambda i: (0, 0)),
        ],
        out_specs=[
            pl.BlockSpec((_BM, D_FEAT), lambda i: (i, 0)),
            pl.BlockSpec((_BM, 1), lambda i: (i, 0)),
        ],
        out_shape=[
            jax.ShapeDtypeStruct((N_NODES, D_FEAT), jnp.float32),
            jax.ShapeDtypeStruct((N_NODES, 1), jnp.float32),
        ],
    )(degA, degB, x, W1)


def _tc_mid(pA, pB, hwp, dis, W, d_out):
    def body(pa, pb, hw, ds, w, out_ref):
        h = jnp.maximum(ds[...] * (pa[...] + pb[...] + hw[...]), 0.0)
        out_ref[...] = ds[...] * jnp.dot(
            h, w[...], preferred_element_type=jnp.float32)

    d_in = hwp.shape[1]
    return pl.pallas_call(
        body,
        grid=(N_NODES // _BM,),
        in_specs=[
            pl.BlockSpec((_BM, d_in), lambda i: (i, 0)),
            pl.BlockSpec((_BM, d_in), lambda i: (i, 0)),
            pl.BlockSpec((_BM, d_in), lambda i: (i, 0)),
            pl.BlockSpec((_BM, 1), lambda i: (i, 0)),
            pl.BlockSpec((d_in, d_out), lambda i: (0, 0)),
        ],
        out_specs=pl.BlockSpec((_BM, d_out), lambda i: (i, 0)),
        out_shape=jax.ShapeDtypeStruct((N_NODES, d_out), jnp.float32),
    )(pA, pB, hwp, dis, W)


def _tc_final(pA, pB, hwp, dis):
    def body(pa, pb, hw, ds, out_ref):
        out_ref[...] = ds[...] * (
            pa[:, :OUT_DIM] + pb[:, :OUT_DIM] + hw[:, :OUT_DIM])

    return pl.pallas_call(
        body,
        grid=(N_NODES // _BM,),
        in_specs=[
            pl.BlockSpec((_BM, D_FEAT), lambda i: (i, 0)),
            pl.BlockSpec((_BM, D_FEAT), lambda i: (i, 0)),
            pl.BlockSpec((_BM, D_FEAT), lambda i: (i, 0)),
            pl.BlockSpec((_BM, 1), lambda i: (i, 0)),
        ],
        out_specs=pl.BlockSpec((_BM, OUT_DIM), lambda i: (i, 0)),
        out_shape=jax.ShapeDtypeStruct((N_NODES, OUT_DIM), jnp.float32),
    )(pA, pB, hwp, dis)


def kernel(x, edge_index, W1, W2, W3):
    src = edge_index[0].astype(jnp.int32)
    dst = edge_index[1].astype(jnp.int32)
    pad = E_PAD - N_EDGES
    srcp = jnp.concatenate([src, jnp.zeros((pad,), jnp.int32)])
    dstp = jnp.concatenate([dst, jnp.full((pad,), N_NODES, jnp.int32)])
    edges3 = jnp.stack(
        [srcp.reshape(TOT_CHUNKS, CHUNK), dstp.reshape(TOT_CHUNKS, CHUNK)],
        axis=1)
    zeros128 = jnp.zeros((N_PAD, D_FEAT), jnp.float32)
    ones128 = jnp.ones((CHUNK, D_FEAT), jnp.float32)
    W3p = jnp.pad(W3, ((0, 0), (0, D_FEAT - OUT_DIM)))

    degP = _deg_kernel(dstp, ones128, zeros128)
    hw1, dis = _tc_first(degP[0], degP[1], x, W1)
    P1 = _spmm128(hw1, edges3, zeros128)
    hw2 = _tc_mid(P1[0], P1[1], hw1, dis, W2, D_FEAT)
    P2 = _spmm128(hw2, edges3, zeros128)
    hw3 = _tc_mid(P2[0], P2[1], hw2, dis, W3p, D_FEAT)
    P3 = _spmm128(hw3, edges3, zeros128)
    return _tc_final(P3[0], P3[1], hw3, dis)


# speedup vs baseline: 2.3811x; 1.0421x over previous
import functools

import jax
import jax.numpy as jnp
from jax import lax
from jax.experimental import pallas as pl
from jax.experimental.pallas import tpu as pltpu
from jax.experimental.pallas import tpu_sc as plsc

N_NODES = 10000
D_FEAT = 128
OUT_DIM = 64
N_EDGES = 320000

NC, NS = 2, 16
NW = NC * NS
CHUNK = 120
TOT_CHUNKS = 2688
E_PAD = TOT_CHUNKS * CHUNK
EPW = E_PAD // NW
N_PAD = 10240
RPT = N_PAD // NS
N_CHUNKS = EPW // CHUNK
CH0 = 129
CH1 = TOT_CHUNKS // NS - CH0

_MESH = plsc.VectorSubcoreMesh(core_axis_name="c", subcore_axis_name="s")


def _make_spmm(d):
    @functools.partial(
        pl.kernel,
        out_type=jax.ShapeDtypeStruct((NC, N_PAD, d), jnp.float32),
        mesh=_MESH,
        scratch_types=[
            pltpu.VMEM((2, CHUNK), jnp.int32),
            pltpu.VMEM((2, CHUNK), jnp.int32),
            pltpu.VMEM((2, CHUNK), jnp.int32),
            pltpu.VMEM((CHUNK, d), jnp.float32),
            pltpu.VMEM((CHUNK, d), jnp.float32),
            pltpu.VMEM((CHUNK, d), jnp.float32),
            pltpu.VMEM_SHARED((N_PAD, d), jnp.float32),
            pltpu.SemaphoreType.DMA,
            pltpu.SemaphoreType.DMA,
            pltpu.SemaphoreType.DMA,
            pltpu.SemaphoreType.DMA,
            pltpu.SemaphoreType.DMA,
            pltpu.SemaphoreType.DMA,
        ],
    )
    def spmm(rows_hbm, edges_hbm, zeros_hbm, out_hbm,
             e0, e1, e2, buf0, buf1, buf2, agg, g0, g1, g2, i0, i1, i2):
        c = lax.axis_index("c")
        s = lax.axis_index("s")
        r0 = s * RPT
        es = (e0, e1, e2)
        bufs = (buf0, buf1, buf2)
        gs = (g0, g1, g2)
        isems = (i0, i1, i2)
        pltpu.sync_copy(zeros_hbm.at[pl.ds(r0, RPT)], agg.at[pl.ds(r0, RPT)])
        plsc.subcore_barrier()

        base = jnp.where(c == 0, s * CH0, NS * CH0 + s * CH1)
        base = jnp.minimum(base, TOT_CHUNKS - 3)
        iters = jnp.where(c == 0, CH0 // 3, CH1 // 3)
        last = base + 3 * iters - 1
        last = jnp.maximum(last, base + 2)

        for k in range(3):
            pltpu.async_copy(edges_hbm.at[base + k], es[k], isems[k])
        for k in range(3):
            pltpu.make_async_copy(edges_hbm.at[base], es[k], isems[k]).wait()
            pltpu.async_copy(rows_hbm.at[es[k].at[0]], bufs[k], gs[k])

        def body(i, carry):
            cb = base + 3 * i
            for k in range(3):
                pltpu.make_async_copy(
                    rows_hbm.at[es[k].at[0]], bufs[k], gs[k]).wait()
                pltpu.sync_copy(bufs[k], agg.at[es[k].at[1]], add=True)
                cn = jnp.minimum(cb + k + 3, last)
                pltpu.async_copy(edges_hbm.at[cn], es[k], isems[k])
                pltpu.make_async_copy(edges_hbm.at[base], es[k], isems[k]).wait()
                pltpu.async_copy(rows_hbm.at[es[k].at[0]], bufs[k], gs[k])
            return carry

        lax.fori_loop(0, iters, body, 0)
        for k in range(3):
            pltpu.make_async_copy(rows_hbm.at[es[k].at[0]], bufs[k], gs[k]).wait()
        plsc.subcore_barrier()
        pltpu.sync_copy(agg.at[pl.ds(r0, RPT)], out_hbm.at[c, pl.ds(r0, RPT)])

    return spmm


_spmm128 = _make_spmm(D_FEAT)


@functools.partial(
    pl.kernel,
    out_type=jax.ShapeDtypeStruct((NC, N_PAD, D_FEAT), jnp.float32),
    mesh=_MESH,
    scratch_types=[
        pltpu.VMEM((CHUNK,), jnp.int32),
        pltpu.VMEM((CHUNK,), jnp.int32),
        pltpu.VMEM((CHUNK, D_FEAT), jnp.float32),
        pltpu.VMEM_SHARED((N_PAD, D_FEAT), jnp.float32),
        pltpu.SemaphoreType.DMA,
        pltpu.SemaphoreType.DMA,
    ],
)
def _deg_kernel(dst_hbm, ones_hbm, zeros_hbm, out_hbm,
                idx0, idx1, ones_v, agg, i0, i1):
    c = lax.axis_index("c")
    s = lax.axis_index("s")
    wid = s * NC + c
    r0 = s * RPT
    base0 = wid * EPW
    last = base0 + (N_CHUNKS - 1) * CHUNK
    pltpu.sync_copy(ones_hbm, ones_v)
    pltpu.sync_copy(zeros_hbm.at[pl.ds(r0, RPT)], agg.at[pl.ds(r0, RPT)])
    plsc.subcore_barrier()

    pltpu.async_copy(dst_hbm.at[pl.ds(base0, CHUNK)], idx0, i0)

    def body(j, carry):
        b0 = base0 + 2 * j * CHUNK
        pltpu.async_copy(dst_hbm.at[pl.ds(b0 + CHUNK, CHUNK)], idx1, i1)
        pltpu.make_async_copy(dst_hbm.at[pl.ds(base0, CHUNK)], idx0, i0).wait()
        pltpu.sync_copy(ones_v, agg.at[idx0], add=True)
        b2 = jnp.minimum(b0 + 2 * CHUNK, last)
        pltpu.async_copy(dst_hbm.at[pl.ds(b2, CHUNK)], idx0, i0)
        pltpu.make_async_copy(dst_hbm.at[pl.ds(base0, CHUNK)], idx1, i1).wait()
        pltpu.sync_copy(ones_v, agg.at[idx1], add=True)
        return carry

    lax.fori_loop(0, N_CHUNKS // 2, body, 0)
    pltpu.make_async_copy(dst_hbm.at[pl.ds(base0, CHUNK)], idx0, i0).wait()
    plsc.subcore_barrier()
    pltpu.sync_copy(agg.at[pl.ds(r0, RPT)], out_hbm.at[c, pl.ds(r0, RPT)])


_BM = 1000


def _tc_first(degA, degB, x, W1):
    def body(da, db, xb, w, hw_ref, dis_ref):
        deg = da[:, :1] + db[:, :1] + 1.0
        dis = lax.rsqrt(jnp.maximum(deg, 1.0))
        hw = jnp.dot(xb[...], w[...], preferred_element_type=jnp.float32)
        hw_ref[...] = dis * hw
        dis_ref[...] = dis

    return pl.pallas_call(
        body,
        grid=(N_NODES // _BM,),
        in_specs=[
            pl.BlockSpec((_BM, D_FEAT), lambda i: (i, 0)),
            pl.BlockSpec((_BM, D_FEAT), lambda i: (i, 0)),
            pl.BlockSpec((_BM, D_FEAT), lambda i: (i, 0)),
            pl.BlockSpec((D_FEAT, D_FEAT), lambda i: (0, 0)),
        ],
        out_specs=[
            pl.BlockSpec((_BM, D_FEAT), lambda i: (i, 0)),
            pl.BlockSpec((_BM, 1), lambda i: (i, 0)),
        ],
        out_shape=[
            jax.ShapeDtypeStruct((N_NODES, D_FEAT), jnp.float32),
            jax.ShapeDtypeStruct((N_NODES, 1), jnp.float32),
        ],
    )(degA, degB, x, W1)


def _tc_mid(pA, pB, hwp, dis, W, d_out):
    def body(pa, pb, hw, ds, w, out_ref):
        h = jnp.maximum(ds[...] * (pa[...] + pb[...] + hw[...]), 0.0)
        out_ref[...] = ds[...] * jnp.dot(
            h, w[...], preferred_element_type=jnp.float32)

    d_in = hwp.shape[1]
    return pl.pallas_call(
        body,
        grid=(N_NODES // _BM,),
        in_specs=[
            pl.BlockSpec((_BM, d_in), lambda i: (i, 0)),
            pl.BlockSpec((_BM, d_in), lambda i: (i, 0)),
            pl.BlockSpec((_BM, d_in), lambda i: (i, 0)),
            pl.BlockSpec((_BM, 1), lambda i: (i, 0)),
            pl.BlockSpec((d_in, d_out), lambda i: (0, 0)),
        ],
        out_specs=pl.BlockSpec((_BM, d_out), lambda i: (i, 0)),
        out_shape=jax.ShapeDtypeStruct((N_NODES, d_out), jnp.float32),
    )(pA, pB, hwp, dis, W)


def _tc_final(pA, pB, hwp, dis):
    def body(pa, pb, hw, ds, out_ref):
        out_ref[...] = ds[...] * (
            pa[:, :OUT_DIM] + pb[:, :OUT_DIM] + hw[:, :OUT_DIM])

    return pl.pallas_call(
        body,
        grid=(N_NODES // _BM,),
        in_specs=[
            pl.BlockSpec((_BM, D_FEAT), lambda i: (i, 0)),
            pl.BlockSpec((_BM, D_FEAT), lambda i: (i, 0)),
            pl.BlockSpec((_BM, D_FEAT), lambda i: (i, 0)),
            pl.BlockSpec((_BM, 1), lambda i: (i, 0)),
        ],
        out_specs=pl.BlockSpec((_BM, OUT_DIM), lambda i: (i, 0)),
        out_shape=jax.ShapeDtypeStruct((N_NODES, OUT_DIM), jnp.float32),
    )(pA, pB, hwp, dis)


def kernel(x, edge_index, W1, W2, W3):
    src = edge_index[0].astype(jnp.int32)
    dst = edge_index[1].astype(jnp.int32)
    pad = E_PAD - N_EDGES
    srcp = jnp.concatenate([src, jnp.zeros((pad,), jnp.int32)])
    dstp = jnp.concatenate([dst, jnp.full((pad,), N_NODES, jnp.int32)])
    edges3 = jnp.stack(
        [srcp.reshape(TOT_CHUNKS, CHUNK), dstp.reshape(TOT_CHUNKS, CHUNK)],
        axis=1)
    zeros128 = jnp.zeros((N_PAD, D_FEAT), jnp.float32)
    ones128 = jnp.ones((CHUNK, D_FEAT), jnp.float32)
    W3p = jnp.pad(W3, ((0, 0), (0, D_FEAT - OUT_DIM)))

    degP = _deg_kernel(dstp, ones128, zeros128)
    hw1, dis = _tc_first(degP[0], degP[1], x, W1)
    P1 = _spmm128(hw1, edges3, zeros128)
    hw2 = _tc_mid(P1[0], P1[1], hw1, dis, W2, D_FEAT)
    P2 = _spmm128(hw2, edges3, zeros128)
    hw3 = _tc_mid(P2[0], P2[1], hw2, dis, W3p, D_FEAT)
    P3 = _spmm128(hw3, edges3, zeros128)
    return _tc_final(P3[0], P3[1], hw3, dis)


# speedup vs baseline: 2.3885x; 1.0031x over previous
import functools

import jax
import jax.numpy as jnp
from jax import lax
from jax.experimental import pallas as pl
from jax.experimental.pallas import tpu as pltpu
from jax.experimental.pallas import tpu_sc as plsc

N_NODES = 10000
D_FEAT = 128
OUT_DIM = 64
N_EDGES = 320000

NC, NS = 2, 16
NW = NC * NS
CHUNK = 120
TOT_CHUNKS = 2688
E_PAD = TOT_CHUNKS * CHUNK
EPW = E_PAD // NW
N_PAD = 10240
RPT = N_PAD // NS
N_CHUNKS = EPW // CHUNK
CH0 = 132
CH1 = TOT_CHUNKS // NS - CH0

_MESH = plsc.VectorSubcoreMesh(core_axis_name="c", subcore_axis_name="s")


def _make_spmm(d):
    @functools.partial(
        pl.kernel,
        out_type=jax.ShapeDtypeStruct((NC, N_PAD, d), jnp.float32),
        mesh=_MESH,
        scratch_types=[
            pltpu.VMEM((2, CHUNK), jnp.int32),
            pltpu.VMEM((2, CHUNK), jnp.int32),
            pltpu.VMEM((2, CHUNK), jnp.int32),
            pltpu.VMEM((CHUNK, d), jnp.float32),
            pltpu.VMEM((CHUNK, d), jnp.float32),
            pltpu.VMEM((CHUNK, d), jnp.float32),
            pltpu.VMEM_SHARED((N_PAD, d), jnp.float32),
            pltpu.SemaphoreType.DMA,
            pltpu.SemaphoreType.DMA,
            pltpu.SemaphoreType.DMA,
            pltpu.SemaphoreType.DMA,
            pltpu.SemaphoreType.DMA,
            pltpu.SemaphoreType.DMA,
        ],
    )
    def spmm(rows_hbm, edges_hbm, zeros_hbm, out_hbm,
             e0, e1, e2, buf0, buf1, buf2, agg, g0, g1, g2, i0, i1, i2):
        c = lax.axis_index("c")
        s = lax.axis_index("s")
        r0 = s * RPT
        es = (e0, e1, e2)
        bufs = (buf0, buf1, buf2)
        gs = (g0, g1, g2)
        isems = (i0, i1, i2)
        pltpu.sync_copy(zeros_hbm.at[pl.ds(r0, RPT)], agg.at[pl.ds(r0, RPT)])
        plsc.subcore_barrier()

        base = jnp.where(c == 0, s * CH0, NS * CH0 + s * CH1)
        base = jnp.minimum(base, TOT_CHUNKS - 3)
        iters = jnp.where(c == 0, CH0 // 3, CH1 // 3)
        last = base + 3 * iters - 1
        last = jnp.maximum(last, base + 2)

        for k in range(3):
            pltpu.async_copy(edges_hbm.at[base + k], es[k], isems[k])
        for k in range(3):
            pltpu.make_async_copy(edges_hbm.at[base], es[k], isems[k]).wait()
            pltpu.async_copy(rows_hbm.at[es[k].at[0]], bufs[k], gs[k])

        def body(i, carry):
            cb = base + 3 * i
            for k in range(3):
                pltpu.make_async_copy(
                    rows_hbm.at[es[k].at[0]], bufs[k], gs[k]).wait()
                pltpu.sync_copy(bufs[k], agg.at[es[k].at[1]], add=True)
                cn = jnp.minimum(cb + k + 3, last)
                pltpu.async_copy(edges_hbm.at[cn], es[k], isems[k])
                pltpu.make_async_copy(edges_hbm.at[base], es[k], isems[k]).wait()
                pltpu.async_copy(rows_hbm.at[es[k].at[0]], bufs[k], gs[k])
            return carry

        lax.fori_loop(0, iters, body, 0)
        for k in range(3):
            pltpu.make_async_copy(rows_hbm.at[es[k].at[0]], bufs[k], gs[k]).wait()
        plsc.subcore_barrier()
        pltpu.sync_copy(agg.at[pl.ds(r0, RPT)], out_hbm.at[c, pl.ds(r0, RPT)])

    return spmm


_spmm128 = _make_spmm(D_FEAT)


@functools.partial(
    pl.kernel,
    out_type=jax.ShapeDtypeStruct((NC, N_PAD, D_FEAT), jnp.float32),
    mesh=_MESH,
    scratch_types=[
        pltpu.VMEM((CHUNK,), jnp.int32),
        pltpu.VMEM((CHUNK,), jnp.int32),
        pltpu.VMEM((CHUNK, D_FEAT), jnp.float32),
        pltpu.VMEM_SHARED((N_PAD, D_FEAT), jnp.float32),
        pltpu.SemaphoreType.DMA,
        pltpu.SemaphoreType.DMA,
    ],
)
def _deg_kernel(dst_hbm, ones_hbm, zeros_hbm, out_hbm,
                idx0, idx1, ones_v, agg, i0, i1):
    c = lax.axis_index("c")
    s = lax.axis_index("s")
    wid = s * NC + c
    r0 = s * RPT
    base0 = wid * EPW
    last = base0 + (N_CHUNKS - 1) * CHUNK
    pltpu.sync_copy(ones_hbm, ones_v)
    pltpu.sync_copy(zeros_hbm.at[pl.ds(r0, RPT)], agg.at[pl.ds(r0, RPT)])
    plsc.subcore_barrier()

    pltpu.async_copy(dst_hbm.at[pl.ds(base0, CHUNK)], idx0, i0)

    def body(j, carry):
        b0 = base0 + 2 * j * CHUNK
        pltpu.async_copy(dst_hbm.at[pl.ds(b0 + CHUNK, CHUNK)], idx1, i1)
        pltpu.make_async_copy(dst_hbm.at[pl.ds(base0, CHUNK)], idx0, i0).wait()
        pltpu.sync_copy(ones_v, agg.at[idx0], add=True)
        b2 = jnp.minimum(b0 + 2 * CHUNK, last)
        pltpu.async_copy(dst_hbm.at[pl.ds(b2, CHUNK)], idx0, i0)
        pltpu.make_async_copy(dst_hbm.at[pl.ds(base0, CHUNK)], idx1, i1).wait()
        pltpu.sync_copy(ones_v, agg.at[idx1], add=True)
        return carry

    lax.fori_loop(0, N_CHUNKS // 2, body, 0)
    pltpu.make_async_copy(dst_hbm.at[pl.ds(base0, CHUNK)], idx0, i0).wait()
    plsc.subcore_barrier()
    pltpu.sync_copy(agg.at[pl.ds(r0, RPT)], out_hbm.at[c, pl.ds(r0, RPT)])


_BM = 1000


def _tc_first(degA, degB, x, W1):
    def body(da, db, xb, w, hw_ref, dis_ref):
        deg = da[:, :1] + db[:, :1] + 1.0
        dis = lax.rsqrt(jnp.maximum(deg, 1.0))
        hw = jnp.dot(xb[...], w[...], preferred_element_type=jnp.float32)
        hw_ref[...] = dis * hw
        dis_ref[...] = dis

    return pl.pallas_call(
        body,
        grid=(N_NODES // _BM,),
        in_specs=[
            pl.BlockSpec((_BM, D_FEAT), lambda i: (i, 0)),
            pl.BlockSpec((_BM, D_FEAT), lambda i: (i, 0)),
            pl.BlockSpec((_BM, D_FEAT), lambda i: (i, 0)),
            pl.BlockSpec((D_FEAT, D_FEAT), lambda i: (0, 0)),
        ],
        out_specs=[
            pl.BlockSpec((_BM, D_FEAT), lambda i: (i, 0)),
            pl.BlockSpec((_BM, 1), lambda i: (i, 0)),
        ],
        out_shape=[
            jax.ShapeDtypeStruct((N_NODES, D_FEAT), jnp.float32),
            jax.ShapeDtypeStruct((N_NODES, 1), jnp.float32),
        ],
    )(degA, degB, x, W1)


def _tc_mid(pA, pB, hwp, dis, W, d_out):
    def body(pa, pb, hw, ds, w, out_ref):
        h = jnp.maximum(ds[...] * (pa[...] + pb[...] + hw[...]), 0.0)
        out_ref[...] = ds[...] * jnp.dot(
            h, w[...], preferred_element_type=jnp.float32)

    d_in = hwp.shape[1]
    return pl.pallas_call(
        body,
        grid=(N_NODES // _BM,),
        in_specs=[
            pl.BlockSpec((_BM, d_in), lambda i: (i, 0)),
            pl.BlockSpec((_BM, d_in), lambda i: (i, 0)),
            pl.BlockSpec((_BM, d_in), lambda i: (i, 0)),
            pl.BlockSpec((_BM, 1), lambda i: (i, 0)),
            pl.BlockSpec((d_in, d_out), lambda i: (0, 0)),
        ],
        out_specs=pl.BlockSpec((_BM, d_out), lambda i: (i, 0)),
        out_shape=jax.ShapeDtypeStruct((N_NODES, d_out), jnp.float32),
    )(pA, pB, hwp, dis, W)


def _tc_final(pA, pB, hwp, dis):
    def body(pa, pb, hw, ds, out_ref):
        out_ref[...] = ds[...] * (
            pa[:, :OUT_DIM] + pb[:, :OUT_DIM] + hw[:, :OUT_DIM])

    return pl.pallas_call(
        body,
        grid=(N_NODES // _BM,),
        in_specs=[
            pl.BlockSpec((_BM, D_FEAT), lambda i: (i, 0)),
            pl.BlockSpec((_BM, D_FEAT), lambda i: (i, 0)),
            pl.BlockSpec((_BM, D_FEAT), lambda i: (i, 0)),
            pl.BlockSpec((_BM, 1), lambda i: (i, 0)),
        ],
        out_specs=pl.BlockSpec((_BM, OUT_DIM), lambda i: (i, 0)),
        out_shape=jax.ShapeDtypeStruct((N_NODES, OUT_DIM), jnp.float32),
    )(pA, pB, hwp, dis)


def kernel(x, edge_index, W1, W2, W3):
    src = edge_index[0].astype(jnp.int32)
    dst = edge_index[1].astype(jnp.int32)
    pad = E_PAD - N_EDGES
    srcp = jnp.concatenate([src, jnp.zeros((pad,), jnp.int32)])
    dstp = jnp.concatenate([dst, jnp.full((pad,), N_NODES, jnp.int32)])
    edges3 = jnp.stack(
        [srcp.reshape(TOT_CHUNKS, CHUNK), dstp.reshape(TOT_CHUNKS, CHUNK)],
        axis=1)
    zeros128 = jnp.zeros((N_PAD, D_FEAT), jnp.float32)
    ones128 = jnp.ones((CHUNK, D_FEAT), jnp.float32)
    W3p = jnp.pad(W3, ((0, 0), (0, D_FEAT - OUT_DIM)))

    degP = _deg_kernel(dstp, ones128, zeros128)
    hw1, dis = _tc_first(degP[0], degP[1], x, W1)
    P1 = _spmm128(hw1, edges3, zeros128)
    hw2 = _tc_mid(P1[0], P1[1], hw1, dis, W2, D_FEAT)
    P2 = _spmm128(hw2, edges3, zeros128)
    hw3 = _tc_mid(P2[0], P2[1], hw2, dis, W3p, D_FEAT)
    P3 = _spmm128(hw3, edges3, zeros128)
    return _tc_final(P3[0], P3[1], hw3, dis)
